# baseline (speedup 1.0000x reference)
import functools

import jax
import jax.numpy as jnp
from jax import lax
from jax.experimental import pallas as pl
from jax.experimental.pallas import tpu as pltpu
from jax.experimental.pallas import tpu_sc as plsc

D = 128
NMIN = 60
NHOUR = 24
NC = 2
NS = 16
NW = NC * NS
L = 16
CHUNK = 128
NFEAT = 5
MIN_CH = 3
HOUR_CH = 2


def _idx_kernel(xt_ref, minute_ref, hour_ref, idx_ref, c_ref):
    @pl.when(pl.program_id(0) == 0)
    def _():
        c_ref[...] = minute_ref[...][:, None, :] + hour_ref[...][None, :, :]

    m = (xt_ref[MIN_CH] * 59.0).astype(jnp.int32)
    h = (xt_ref[HOUR_CH] * 23.0).astype(jnp.int32)
    idx_ref[0] = m * NHOUR + h


def _token_idx(x_mark, minute_embed, hour_embed):
    b, t, _ = x_mark.shape
    xt = jnp.transpose(x_mark, (2, 1, 0))
    idx, c = pl.pallas_call(
        _idx_kernel,
        grid=(b // CHUNK,),
        in_specs=[
            pl.BlockSpec((NFEAT, t, CHUNK), lambda g: (0, 0, g)),
            pl.BlockSpec((NMIN, D), lambda g: (0, 0)),
            pl.BlockSpec((NHOUR, D), lambda g: (0, 0)),
        ],
        out_specs=[
            pl.BlockSpec((1, t, CHUNK), lambda g: (g, 0, 0)),
            pl.BlockSpec((NMIN, NHOUR, D), lambda g: (0, 0, 0)),
        ],
        out_shape=[
            jax.ShapeDtypeStruct((b // CHUNK, t, CHUNK), jnp.int32),
            jax.ShapeDtypeStruct((NMIN, NHOUR, D), jnp.float32),
        ],
    )(xt, minute_embed, hour_embed)
    return idx, c.reshape(NMIN * NHOUR, D)


def _make_gather(n_b, n_t):
    assert n_b == NW * CHUNK
    n_tok = n_b * n_t
    mesh = plsc.VectorSubcoreMesh(
        core_axis_name="c", subcore_axis_name="s", num_cores=NC, num_subcores=NS
    )

    @functools.partial(
        pl.kernel,
        out_type=jax.ShapeDtypeStruct((n_tok, D), jnp.float32),
        mesh=mesh,
        scratch_types=(
            [pltpu.VMEM((n_t, CHUNK), jnp.int32)]
            + [pltpu.VMEM((CHUNK,), jnp.int32) for _ in range(5)]
            + [pltpu.VMEM((CHUNK, D), jnp.float32) for _ in range(5)]
            + [pltpu.SemaphoreType.DMA for _ in range(10)]
            + [pltpu.VMEM_SHARED((NMIN * NHOUR, D), jnp.float32)]
        ),
        compiler_params=pltpu.CompilerParams(needs_layout_passes=False),
    )
    def gather(idx_hbm, c_hbm, out_hbm, slab,
               i0, i1, i2, i3, i4, r0, r1, r2, r3, r4,
               gs0, gs1, gs2, gs3, gs4, ss0, ss1, ss2, ss3, ss4, c_sp):
        ib = [i0, i1, i2, i3, i4]
        rb = [r0, r1, r2, r3, r4]
        gs = [gs0, gs1, gs2, gs3, gs4]
        ss = [ss0, ss1, ss2, ss3, ss4]
        wid = lax.axis_index("s") * NC + lax.axis_index("c")
        w_base = wid * n_t * CHUNK

        @pl.when(lax.axis_index("s") == 0)
        def _():
            pltpu.sync_copy(c_hbm, c_sp)

        pltpu.sync_copy(idx_hbm.at[wid], slab)
        plsc.subcore_barrier()

        def fire(ri, ib, rows, gsem):
            for jj in range(CHUNK // L):
                q = lax.iota(jnp.int32, L) + (CHUNK * ri + L * jj)
                b_loc = q // n_t
                t = q - b_loc * n_t
                ib[pl.ds(L * jj, L)] = plsc.load_gather(slab, [t, b_loc])
            pltpu.async_copy(c_sp.at[ib], rows, gsem)

        def wait_g(ib, rows, gsem):
            pltpu.make_async_copy(c_sp.at[ib], rows, gsem).wait()

        def scatter(ti, rows, ssem):
            pltpu.async_copy(
                rows, out_hbm.at[pl.ds(w_base + ti * CHUNK, CHUNK)], ssem
            )

        def wait_s(ti, rows, ssem):
            pltpu.make_async_copy(
                rows, out_hbm.at[pl.ds(w_base + ti * CHUNK, CHUNK)], ssem
            ).wait()

        NB = 5
        n_chunks = n_t
        n_groups = n_chunks // NB
        for k in range(NB - 1):
            fire(k, ib[k], rb[k], gs[k])

        def body(g, carry):
            c0 = NB * g
            for k in range(NB):
                c = c0 + k
                s3 = (k + NB - 1) % NB
                wait_g(ib[k], rb[k], gs[k])
                scatter(c, rb[k], ss[k])

                @pl.when(c + NB - 1 < n_chunks)
                def _():
                    @pl.when(c >= 1)
                    def _():
                        wait_s(c - 1, rb[s3], ss[s3])

                    fire(c + NB - 1, ib[s3], rb[s3], gs[s3])

            return carry

        lax.fori_loop(0, n_groups, body, 0)
        for j in range(NB):
            c = n_chunks - NB + j
            wait_s(c, rb[c % NB], ss[c % NB])

    return gather


def kernel(x_mark, minute_embed, hour_embed):
    b, t, _ = x_mark.shape
    idx, c_table = _token_idx(x_mark, minute_embed, hour_embed)
    out = _make_gather(b, t)(idx, c_table)
    return out.reshape(b, t, D)
